# baseline (speedup 1.0000x reference)
import functools

import jax
import jax.numpy as jnp
from jax import lax
from jax.experimental import pallas as pl
from jax.experimental.pallas import tpu as pltpu
from jax.experimental.pallas import tpu_sc as plsc

_B, _H, _W, _C = 16, 16, 16, 768
_N = 4096
_ROWS = _B * _N
_NW = 32
_CH2 = _C // 2
_CK = 16
_NCK = _N // _CK
_SPH = 1024
_mesh = plsc.VectorSubcoreMesh(core_axis_name="c", subcore_axis_name="s")


@functools.partial(
    pl.kernel,
    mesh=_mesh,
    out_type=jax.ShapeDtypeStruct((_ROWS, _C), jnp.float32),
    compiler_params=pltpu.CompilerParams(
        needs_layout_passes=False,
        skip_device_barrier=True,
        disable_bounds_checks=True,
        disable_semaphore_checks=True,
    ),
    scratch_types=[
        pltpu.VMEM((64, 128), jnp.int32),
        pltpu.VMEM((_N,), jnp.int32),
        pltpu.VMEM((256, _CH2), jnp.float32),
        pltpu.VMEM((_CK, _CH2), jnp.float32),
        pltpu.VMEM((_CK, _CH2), jnp.float32),
        pltpu.SemaphoreType.DMA,
        pltpu.SemaphoreType.DMA,
        pltpu.SemaphoreType.DMA,
    ],
)
def _sc_gather(fmap_hbm, idx_hbm, out_hbm, raw_v, rid_v, subtab,
               buf0, buf1, ssem0, ssem1, tsem):
    table_hbm = fmap_hbm.reshape(_B * _H * _W, _C)
    wid = lax.axis_index("s") * 2 + lax.axis_index("c")
    batch = wid // 2
    half = wid % 2
    cstart = half * _CH2
    obase = batch * _N

    stage = pltpu.async_copy(
        table_hbm.at[pl.ds(batch * (_H * _W), _H * _W), pl.ds(cstart, _CH2)],
        subtab, tsem,
    )

    pltpu.sync_copy(idx_hbm.at[pl.ds(batch * 64, 64), :], raw_v)

    def transform(k, carry):
        r = 2 * (k // 8)
        col = (k % 8) * 16
        i = raw_v[r, pl.ds(col, 16)]
        j = raw_v[r + 1, pl.ds(col, 16)]
        rid_v[pl.ds(k * 16, 16)] = (i >> 5) * _W + (j >> 5)
        return carry

    lax.fori_loop(0, _N // 16, transform, 0)
    stage.wait()

    bufs = (buf0, buf1)
    ssems = (ssem0, ssem1)

    def fill(buf, cbase):
        rv = rid_v[pl.ds(cbase * _CK, _CK)]
        for bn in range(_CK):
            r = rv[bn]
            for k in range(_CH2 // 16):
                buf[bn, pl.ds(k * 16, 16)] = subtab[r, pl.ds(k * 16, 16)]

    def start_store(buf, c, sem):
        return pltpu.async_copy(
            buf, out_hbm.at[pl.ds(obase + c * _CK, _CK), pl.ds(cstart, _CH2)],
            sem,
        )

    def pair(cc, carry):
        c0 = 2 * cc

        @pl.when(cc > 0)
        def _():
            pltpu.make_async_copy(
                buf0,
                out_hbm.at[pl.ds(obase, _CK), pl.ds(cstart, _CH2)],
                ssem0,
            ).wait()

        fill(buf0, c0)
        start_store(buf0, c0, ssem0)

        @pl.when(cc > 0)
        def _():
            pltpu.make_async_copy(
                buf1,
                out_hbm.at[pl.ds(obase, _CK), pl.ds(cstart, _CH2)],
                ssem1,
            ).wait()

        fill(buf1, c0 + 1)
        start_store(buf1, c0 + 1, ssem1)
        return carry

    lax.fori_loop(0, _NCK // 2, pair, 0)

    pltpu.make_async_copy(
        buf0, out_hbm.at[pl.ds(obase, _CK), pl.ds(cstart, _CH2)], ssem0
    ).wait()
    pltpu.make_async_copy(
        buf1, out_hbm.at[pl.ds(obase, _CK), pl.ds(cstart, _CH2)], ssem1
    ).wait()


def kernel(fmap, idx):
    idx_planar = (
        idx.astype(jnp.int32)
        .reshape(_B, _N // 128, 128, 2)
        .transpose(0, 1, 3, 2)
        .reshape(_B * (_N // 128) * 2, 128)
    )
    out = _sc_gather(fmap, idx_planar)
    return out.reshape(_B, _N, _C)


# speedup vs baseline: 2.7194x; 2.7194x over previous
import functools

import jax
import jax.numpy as jnp
from jax import lax
from jax.experimental import pallas as pl
from jax.experimental.pallas import tpu as pltpu
from jax.experimental.pallas import tpu_sc as plsc

_B, _H, _W, _C = 16, 16, 16, 768
_N = 4096
_ROWS = _B * _N
_NW = 32
_RPW = _ROWS // _NW
_CH = 32
_NCH = _RPW // _CH

_mesh = plsc.VectorSubcoreMesh(core_axis_name="c", subcore_axis_name="s")


@functools.partial(
    pl.kernel,
    mesh=_mesh,
    out_type=jax.ShapeDtypeStruct((_ROWS, _C), jnp.float32),
    compiler_params=pltpu.CompilerParams(
        needs_layout_passes=False,
        skip_device_barrier=True,
        disable_bounds_checks=True,
        disable_semaphore_checks=True,
    ),
    scratch_types=[
        pltpu.VMEM((32, 128), jnp.int32),
        pltpu.VMEM((_RPW,), jnp.int32),
        pltpu.VMEM((_CH, _C), jnp.float32),
        pltpu.VMEM((_CH, _C), jnp.float32),
        pltpu.VMEM((_CH, _C), jnp.float32),
        pltpu.VMEM((_CH, _C), jnp.float32),
        pltpu.SemaphoreType.DMA,
        pltpu.SemaphoreType.DMA,
        pltpu.SemaphoreType.DMA,
        pltpu.SemaphoreType.DMA,
        pltpu.SemaphoreType.DMA,
        pltpu.SemaphoreType.DMA,
        pltpu.SemaphoreType.DMA,
        pltpu.SemaphoreType.DMA,
    ],
)
def _sc_gather(fmap_hbm, idx_hbm, out_hbm, raw_v, rid_v, buf0, buf1, buf2,
               buf3, gsem0, gsem1, gsem2, gsem3, ssem0, ssem1, ssem2, ssem3):
    table_hbm = fmap_hbm.reshape(_B * _H * _W, _C)
    wid = lax.axis_index("s") * 2 + lax.axis_index("c")
    base_row = wid * _RPW
    bval = (wid // 2) * (_H * _W)

    pltpu.sync_copy(idx_hbm.at[pl.ds(wid * 32, 32), :], raw_v)

    def transform(k, carry):
        r = 2 * (k // 8)
        col = (k % 8) * 16
        i = raw_v[r, pl.ds(col, 16)]
        j = raw_v[r + 1, pl.ds(col, 16)]
        rid_v[pl.ds(k * 16, 16)] = (i >> 5) * _W + (j >> 5) + bval
        return carry

    lax.fori_loop(0, _RPW // 16, transform, 0)

    bufs = (buf0, buf1, buf2, buf3)
    gsems = (gsem0, gsem1, gsem2, gsem3)
    ssems = (ssem0, ssem1, ssem2, ssem3)

    def start_gather(c):
        return pltpu.async_copy(
            table_hbm.at[rid_v.at[pl.ds(c * _CH, _CH)]], bufs[c % 4],
            gsems[c % 4],
        )

    def start_store(c):
        return pltpu.async_copy(
            bufs[c % 4], out_hbm.at[pl.ds(base_row + c * _CH, _CH)],
            ssems[c % 4],
        )

    g = [None] * 4
    s = [None] * 4
    for c in range(_NCH):
        p = c % 4
        if s[p] is not None:
            s[p].wait()
        g[p] = start_gather(c)
        if c >= 1:
            q = (c - 1) % 4
            g[q].wait()
            s[q] = start_store(c - 1)
    q = (_NCH - 1) % 4
    g[q].wait()
    s[q] = start_store(_NCH - 1)
    for d in s:
        d.wait()


def kernel(fmap, idx):
    idx_planar = (
        idx.astype(jnp.int32)
        .reshape(_B, _N // 128, 128, 2)
        .transpose(0, 1, 3, 2)
        .reshape(_B * (_N // 128) * 2, 128)
    )
    out = _sc_gather(fmap, idx_planar)
    return out.reshape(_B, _N, _C)
